# Initial kernel scaffold; baseline (speedup 1.0000x reference)
import jax
import jax.numpy as jnp
from jax.experimental import pallas as pl


def kernel(x):
    raise NotImplementedError("write your pallas kernel here")


# speedup vs baseline: 1.1177x; 1.1177x over previous
import functools

import jax
import jax.numpy as jnp
from jax.experimental import pallas as pl
from jax.experimental.pallas import tpu as pltpu

WIN = 8
KEEP_RATIO = 0.5
MIN_KEEP = 1


def _pool_mats(H, W, NH, NW, dtype=jnp.float32):
    r = jax.lax.broadcasted_iota(jnp.int32, (NH, H), 0)
    c = jax.lax.broadcasted_iota(jnp.int32, (NH, H), 1)
    eh = (c // WIN == r).astype(dtype)
    r2 = jax.lax.broadcasted_iota(jnp.int32, (W, NW), 0)
    c2 = jax.lax.broadcasted_iota(jnp.int32, (W, NW), 1)
    ew = (r2 // WIN == c2).astype(dtype)
    return eh, ew


def _energy_kernel(x_ref, pooled_ref, acc_ref, *, nc, C, H, W, NH, NW):
    c = pl.program_id(1)
    part = jnp.sum(jnp.abs(x_ref[0]), axis=0)

    @pl.when(c == 0)
    def _():
        acc_ref[...] = part

    @pl.when(c > 0)
    def _():
        acc_ref[...] += part

    @pl.when(c == nc - 1)
    def _():
        energy = acc_ref[...] / jnp.float32(C)
        eh, ew = _pool_mats(H, W, NH, NW)
        wsum = jax.lax.dot_general(
            energy, ew, (((1,), (0,)), ((), ())),
            precision=jax.lax.Precision.HIGHEST,
            preferred_element_type=jnp.float32)
        hsum = jax.lax.dot_general(
            eh, wsum, (((1,), (0,)), ((), ())),
            precision=jax.lax.Precision.HIGHEST,
            preferred_element_type=jnp.float32)
        pooled_ref[0] = hsum / jnp.float32(WIN * WIN)


def _select_kernel(pooled_ref, wm_ref, tm_ref, *, keep, BT, H, W, NH, NW):
    N = NH * NW
    p = pooled_ref[...]
    pi = jax.lax.bitcast_convert_type(p, jnp.int32)

    def cnt_ge(t):
        return jnp.sum((pi >= t).astype(jnp.int32), axis=(1, 2), keepdims=True)

    def body(_, lohi):
        lo, hi = lohi
        mid = lo + ((hi - lo) // 2) + ((hi - lo) & 1)
        ok = cnt_ge(mid) >= keep
        return (jnp.where(ok, mid, lo), jnp.where(ok, hi, mid - 1))

    lo0 = jnp.zeros((BT, 1, 1), jnp.int32)
    hi0 = jnp.full((BT, 1, 1), 0x7F800000, jnp.int32)
    vstar, _ = jax.lax.fori_loop(0, 31, body, (lo0, hi0))

    gt = pi > vstar
    eq = pi == vstar
    need = keep - jnp.sum(gt.astype(jnp.int32), axis=(1, 2), keepdims=True)

    flat = (jax.lax.broadcasted_iota(jnp.int32, (BT, NH, NW), 1) * NW
            + jax.lax.broadcasted_iota(jnp.int32, (BT, NH, NW), 2))

    def cnt_eq_lt(m):
        return jnp.sum((eq & (flat < m)).astype(jnp.int32), axis=(1, 2),
                       keepdims=True)

    def body2(_, lohi):
        lo, hi = lohi
        mid = (lo + hi) // 2
        ok = cnt_eq_lt(mid) >= need
        return (jnp.where(ok, lo, mid + 1), jnp.where(ok, mid, hi))

    lo0 = jnp.full((BT, 1, 1), 1, jnp.int32)
    hi0 = jnp.full((BT, 1, 1), N, jnp.int32)
    mcut, _ = jax.lax.fori_loop(0, 10, body2, (lo0, hi0))

    wmask = (gt | (eq & (flat < mcut))).astype(jnp.float32)
    wm_ref[...] = wmask

    eh, ew = _pool_mats(H, W, NH, NW)
    for f in range(BT):
        t1 = jax.lax.dot_general(
            eh, wmask[f], (((0,), (0,)), ((), ())),
            preferred_element_type=jnp.float32)
        tm_ref[f] = jax.lax.dot_general(
            t1, ew, (((1,), (1,)), ((), ())),
            preferred_element_type=jnp.float32)


def _apply_kernel(tm_ref, x_ref, out_ref):
    out_ref[0] = x_ref[0] * tm_ref[...]


def kernel(x):
    B, T, C, H, W = x.shape
    assert H % WIN == 0 and W % WIN == 0
    NH, NW = H // WIN, W // WIN
    N = NH * NW
    keep = min(max(MIN_KEEP, int(N * KEEP_RATIO)), N)
    BT = B * T
    x4 = x.reshape(BT, C, H, W)

    CBLK = 16
    nc = C // CBLK

    pooled = pl.pallas_call(
        functools.partial(_energy_kernel, nc=nc, C=C, H=H, W=W, NH=NH, NW=NW),
        grid=(BT, nc),
        in_specs=[pl.BlockSpec((1, CBLK, H, W), lambda bt, c: (bt, c, 0, 0))],
        out_specs=pl.BlockSpec((1, NH, NW), lambda bt, c: (bt, 0, 0)),
        out_shape=jax.ShapeDtypeStruct((BT, NH, NW), jnp.float32),
        scratch_shapes=[pltpu.VMEM((H, W), jnp.float32)],
        compiler_params=pltpu.CompilerParams(
            dimension_semantics=("arbitrary", "arbitrary")),
    )(x4)

    wm, tm = pl.pallas_call(
        functools.partial(_select_kernel, keep=keep, BT=BT, H=H, W=W,
                          NH=NH, NW=NW),
        out_shape=[
            jax.ShapeDtypeStruct((BT, NH, NW), jnp.float32),
            jax.ShapeDtypeStruct((BT, H, W), jnp.float32),
        ],
    )(pooled)

    out = pl.pallas_call(
        _apply_kernel,
        grid=(BT, nc),
        in_specs=[
            pl.BlockSpec((1, H, W), lambda bt, c: (bt, 0, 0)),
            pl.BlockSpec((1, CBLK, H, W), lambda bt, c: (bt, c, 0, 0)),
        ],
        out_specs=pl.BlockSpec((1, CBLK, H, W), lambda bt, c: (bt, c, 0, 0)),
        out_shape=jax.ShapeDtypeStruct((BT, C, H, W), jnp.float32),
        compiler_params=pltpu.CompilerParams(
            dimension_semantics=("arbitrary", "arbitrary")),
    )(tm, x4)

    pruned = out.reshape(B, T, C, H, W)
    token_mask = tm.reshape(B, T, H, W).astype(jnp.bool_)
    window_mask = wm.reshape(B, T, NH, NW).astype(jnp.bool_)
    return (pruned, token_mask, window_mask)


# speedup vs baseline: 1.2306x; 1.1010x over previous
import functools

import jax
import jax.numpy as jnp
from jax.experimental import pallas as pl
from jax.experimental.pallas import tpu as pltpu

WIN = 8
KEEP_RATIO = 0.5
MIN_KEEP = 1


def _pool_mats(H, W, NH, NW, dtype=jnp.float32):
    r = jax.lax.broadcasted_iota(jnp.int32, (NH, H), 0)
    c = jax.lax.broadcasted_iota(jnp.int32, (NH, H), 1)
    eh = (c // WIN == r).astype(dtype)
    r2 = jax.lax.broadcasted_iota(jnp.int32, (W, NW), 0)
    c2 = jax.lax.broadcasted_iota(jnp.int32, (W, NW), 1)
    ew = (r2 // WIN == c2).astype(dtype)
    return eh, ew


def _energy_kernel(x_ref, pooled_ref, acc_ref, *, nc, C, H, W, NH, NW):
    c = pl.program_id(1)
    cblk = x_ref.shape[1]
    part0 = jnp.sum(jnp.abs(x_ref[0, 0:16]), axis=0)

    @pl.when(c == 0)
    def _():
        acc_ref[...] = part0

    @pl.when(c > 0)
    def _():
        acc_ref[...] += part0

    for k in range(16, cblk, 16):
        acc_ref[...] += jnp.sum(jnp.abs(x_ref[0, k:k + 16]), axis=0)

    @pl.when(c == nc - 1)
    def _():
        energy = acc_ref[...] / jnp.float32(C)
        eh, ew = _pool_mats(H, W, NH, NW)
        wsum = jax.lax.dot_general(
            energy, ew, (((1,), (0,)), ((), ())),
            precision=jax.lax.Precision.HIGHEST,
            preferred_element_type=jnp.float32)
        hsum = jax.lax.dot_general(
            eh, wsum, (((1,), (0,)), ((), ())),
            precision=jax.lax.Precision.HIGHEST,
            preferred_element_type=jnp.float32)
        pooled_ref[0] = hsum / jnp.float32(WIN * WIN)


def _select_kernel(pooled_ref, wm_ref, tm_ref, *, keep, BT, H, W, NH, NW):
    N = NH * NW
    p = pooled_ref[...]
    pi = jax.lax.bitcast_convert_type(p, jnp.int32)

    def cnt_ge(t):
        return jnp.sum((pi >= t).astype(jnp.int32), axis=(1, 2), keepdims=True)

    def body(_, lohi):
        lo, hi = lohi
        mid = lo + ((hi - lo) // 2) + ((hi - lo) & 1)
        ok = cnt_ge(mid) >= keep
        return (jnp.where(ok, mid, lo), jnp.where(ok, hi, mid - 1))

    lo0 = jnp.zeros((BT, 1, 1), jnp.int32)
    hi0 = jnp.full((BT, 1, 1), 0x7F800000, jnp.int32)
    vstar, _ = jax.lax.fori_loop(0, 31, body, (lo0, hi0))

    gt = pi > vstar
    eq = pi == vstar
    need = keep - jnp.sum(gt.astype(jnp.int32), axis=(1, 2), keepdims=True)

    flat = (jax.lax.broadcasted_iota(jnp.int32, (BT, NH, NW), 1) * NW
            + jax.lax.broadcasted_iota(jnp.int32, (BT, NH, NW), 2))

    def cnt_eq_lt(m):
        return jnp.sum((eq & (flat < m)).astype(jnp.int32), axis=(1, 2),
                       keepdims=True)

    def body2(_, lohi):
        lo, hi = lohi
        mid = (lo + hi) // 2
        ok = cnt_eq_lt(mid) >= need
        return (jnp.where(ok, lo, mid + 1), jnp.where(ok, mid, hi))

    lo0 = jnp.full((BT, 1, 1), 1, jnp.int32)
    hi0 = jnp.full((BT, 1, 1), N, jnp.int32)
    mcut, _ = jax.lax.fori_loop(0, 10, body2, (lo0, hi0))

    wmask = (gt | (eq & (flat < mcut))).astype(jnp.float32)
    wm_ref[...] = wmask

    eh, ew = _pool_mats(H, W, NH, NW)
    for f in range(BT):
        t1 = jax.lax.dot_general(
            eh, wmask[f], (((0,), (0,)), ((), ())),
            preferred_element_type=jnp.float32)
        tm_ref[f] = jax.lax.dot_general(
            t1, ew, (((1,), (1,)), ((), ())),
            preferred_element_type=jnp.float32)


def _apply_kernel(tm_ref, x_ref, out_ref):
    out_ref[0] = x_ref[0] * tm_ref[...]


def kernel(x):
    B, T, C, H, W = x.shape
    assert H % WIN == 0 and W % WIN == 0
    NH, NW = H // WIN, W // WIN
    N = NH * NW
    keep = min(max(MIN_KEEP, int(N * KEEP_RATIO)), N)
    BT = B * T
    x4 = x.reshape(BT, C, H, W)

    CBLK = 48
    nc = C // CBLK

    pooled = pl.pallas_call(
        functools.partial(_energy_kernel, nc=nc, C=C, H=H, W=W, NH=NH, NW=NW),
        grid=(BT, nc),
        in_specs=[pl.BlockSpec((1, CBLK, H, W), lambda bt, c: (bt, c, 0, 0))],
        out_specs=pl.BlockSpec((1, NH, NW), lambda bt, c: (bt, 0, 0)),
        out_shape=jax.ShapeDtypeStruct((BT, NH, NW), jnp.float32),
        scratch_shapes=[pltpu.VMEM((H, W), jnp.float32)],
        compiler_params=pltpu.CompilerParams(
            dimension_semantics=("arbitrary", "arbitrary")),
    )(x4)

    wm, tm = pl.pallas_call(
        functools.partial(_select_kernel, keep=keep, BT=BT, H=H, W=W,
                          NH=NH, NW=NW),
        out_shape=[
            jax.ShapeDtypeStruct((BT, NH, NW), jnp.float32),
            jax.ShapeDtypeStruct((BT, H, W), jnp.float32),
        ],
    )(pooled)

    out = pl.pallas_call(
        _apply_kernel,
        grid=(BT, nc),
        in_specs=[
            pl.BlockSpec((1, H, W), lambda bt, c: (bt, 0, 0)),
            pl.BlockSpec((1, CBLK, H, W), lambda bt, c: (bt, c, 0, 0)),
        ],
        out_specs=pl.BlockSpec((1, CBLK, H, W), lambda bt, c: (bt, c, 0, 0)),
        out_shape=jax.ShapeDtypeStruct((BT, C, H, W), jnp.float32),
        compiler_params=pltpu.CompilerParams(
            dimension_semantics=("arbitrary", "arbitrary")),
    )(tm, x4)

    pruned = out.reshape(B, T, C, H, W)
    token_mask = tm.reshape(B, T, H, W).astype(jnp.bool_)
    window_mask = wm.reshape(B, T, NH, NW).astype(jnp.bool_)
    return (pruned, token_mask, window_mask)


# speedup vs baseline: 1.2309x; 1.0002x over previous
import functools

import jax
import jax.numpy as jnp
from jax.experimental import pallas as pl
from jax.experimental.pallas import tpu as pltpu

WIN = 8
KEEP_RATIO = 0.5
MIN_KEEP = 1


def _pool_mats(H, W, NH, NW, dtype=jnp.float32):
    r = jax.lax.broadcasted_iota(jnp.int32, (NH, H), 0)
    c = jax.lax.broadcasted_iota(jnp.int32, (NH, H), 1)
    eh = (c // WIN == r).astype(dtype)
    r2 = jax.lax.broadcasted_iota(jnp.int32, (W, NW), 0)
    c2 = jax.lax.broadcasted_iota(jnp.int32, (W, NW), 1)
    ew = (r2 // WIN == c2).astype(dtype)
    return eh, ew


def _energy_kernel(x_ref, pooled_ref, acc_ref, *, nc, C, H, W, NH, NW):
    c = pl.program_id(1)
    cblk = x_ref.shape[1]
    part0 = jnp.sum(jnp.abs(x_ref[0, 0:16]), axis=0)

    @pl.when(c == 0)
    def _():
        acc_ref[...] = part0

    @pl.when(c > 0)
    def _():
        acc_ref[...] += part0

    for k in range(16, cblk, 16):
        acc_ref[...] += jnp.sum(jnp.abs(x_ref[0, k:k + 16]), axis=0)

    @pl.when(c == nc - 1)
    def _():
        energy = acc_ref[...] / jnp.float32(C)
        eh, ew = _pool_mats(H, W, NH, NW)
        wsum = jax.lax.dot_general(
            energy, ew, (((1,), (0,)), ((), ())),
            precision=jax.lax.Precision.HIGHEST,
            preferred_element_type=jnp.float32)
        hsum = jax.lax.dot_general(
            eh, wsum, (((1,), (0,)), ((), ())),
            precision=jax.lax.Precision.HIGHEST,
            preferred_element_type=jnp.float32)
        pooled_ref[0] = hsum / jnp.float32(WIN * WIN)


def _select_kernel(pooled_ref, wm_ref, tm_ref, *, keep, BT, H, W, NH, NW):
    N = NH * NW
    p = pooled_ref[...]
    pi = jax.lax.bitcast_convert_type(p, jnp.int32)

    def cnt_ge(t):
        return jnp.sum((pi >= t).astype(jnp.int32), axis=(1, 2), keepdims=True)

    def body(_, lohi):
        lo, hi = lohi
        mid = lo + ((hi - lo) // 2) + ((hi - lo) & 1)
        ok = cnt_ge(mid) >= keep
        return (jnp.where(ok, mid, lo), jnp.where(ok, hi, mid - 1))

    lo0 = jnp.zeros((BT, 1, 1), jnp.int32)
    hi0 = jnp.full((BT, 1, 1), 0x7F800000, jnp.int32)
    vstar, _ = jax.lax.fori_loop(0, 31, body, (lo0, hi0))

    gt = pi > vstar
    eq = pi == vstar
    need = keep - jnp.sum(gt.astype(jnp.int32), axis=(1, 2), keepdims=True)

    flat = (jax.lax.broadcasted_iota(jnp.int32, (BT, NH, NW), 1) * NW
            + jax.lax.broadcasted_iota(jnp.int32, (BT, NH, NW), 2))

    def cnt_eq_lt(m):
        return jnp.sum((eq & (flat < m)).astype(jnp.int32), axis=(1, 2),
                       keepdims=True)

    def body2(_, lohi):
        lo, hi = lohi
        mid = (lo + hi) // 2
        ok = cnt_eq_lt(mid) >= need
        return (jnp.where(ok, lo, mid + 1), jnp.where(ok, mid, hi))

    lo0 = jnp.full((BT, 1, 1), 1, jnp.int32)
    hi0 = jnp.full((BT, 1, 1), N, jnp.int32)
    mcut, _ = jax.lax.fori_loop(0, 10, body2, (lo0, hi0))

    wmask = (gt | (eq & (flat < mcut))).astype(jnp.float32)
    wm_ref[...] = wmask

    eh, ew = _pool_mats(H, W, NH, NW)
    for f in range(BT):
        t1 = jax.lax.dot_general(
            eh, wmask[f], (((0,), (0,)), ((), ())),
            preferred_element_type=jnp.float32)
        tm_ref[f] = jax.lax.dot_general(
            t1, ew, (((1,), (1,)), ((), ())),
            preferred_element_type=jnp.float32)


def _apply_kernel(tm_ref, x_ref, out_ref):
    out_ref[0] = x_ref[0] * tm_ref[...]


def kernel(x):
    B, T, C, H, W = x.shape
    assert H % WIN == 0 and W % WIN == 0
    NH, NW = H // WIN, W // WIN
    N = NH * NW
    keep = min(max(MIN_KEEP, int(N * KEEP_RATIO)), N)
    BT = B * T
    x4 = x.reshape(BT, C, H, W)

    CBLK = 64
    nc = C // CBLK

    pooled = pl.pallas_call(
        functools.partial(_energy_kernel, nc=nc, C=C, H=H, W=W, NH=NH, NW=NW),
        grid=(BT, nc),
        in_specs=[pl.BlockSpec((1, CBLK, H, W), lambda bt, c: (bt, c, 0, 0))],
        out_specs=pl.BlockSpec((1, NH, NW), lambda bt, c: (bt, 0, 0)),
        out_shape=jax.ShapeDtypeStruct((BT, NH, NW), jnp.float32),
        scratch_shapes=[pltpu.VMEM((H, W), jnp.float32)],
        compiler_params=pltpu.CompilerParams(
            dimension_semantics=("arbitrary", "arbitrary")),
    )(x4)

    wm, tm = pl.pallas_call(
        functools.partial(_select_kernel, keep=keep, BT=BT, H=H, W=W,
                          NH=NH, NW=NW),
        out_shape=[
            jax.ShapeDtypeStruct((BT, NH, NW), jnp.float32),
            jax.ShapeDtypeStruct((BT, H, W), jnp.float32),
        ],
    )(pooled)

    out = pl.pallas_call(
        _apply_kernel,
        grid=(BT, nc),
        in_specs=[
            pl.BlockSpec((1, H, W), lambda bt, c: (bt, 0, 0)),
            pl.BlockSpec((1, CBLK, H, W), lambda bt, c: (bt, c, 0, 0)),
        ],
        out_specs=pl.BlockSpec((1, CBLK, H, W), lambda bt, c: (bt, c, 0, 0)),
        out_shape=jax.ShapeDtypeStruct((BT, C, H, W), jnp.float32),
        compiler_params=pltpu.CompilerParams(
            dimension_semantics=("arbitrary", "arbitrary"),
            vmem_limit_bytes=100 * 1024 * 1024),
    )(tm, x4)

    pruned = out.reshape(B, T, C, H, W)
    token_mask = tm.reshape(B, T, H, W).astype(jnp.bool_)
    window_mask = wm.reshape(B, T, NH, NW).astype(jnp.bool_)
    return (pruned, token_mask, window_mask)


# speedup vs baseline: 1.4227x; 1.1559x over previous
import functools

import jax
import jax.numpy as jnp
from jax.experimental import pallas as pl
from jax.experimental.pallas import tpu as pltpu

WIN = 8
KEEP_RATIO = 0.5
MIN_KEEP = 1

_ANY = pl.ANY


def _pool_mats(H, W, NH, NW, dtype=jnp.float32):
    r = jax.lax.broadcasted_iota(jnp.int32, (NH, H), 0)
    c = jax.lax.broadcasted_iota(jnp.int32, (NH, H), 1)
    eh = (c // WIN == r).astype(dtype)
    r2 = jax.lax.broadcasted_iota(jnp.int32, (W, NW), 0)
    c2 = jax.lax.broadcasted_iota(jnp.int32, (W, NW), 1)
    ew = (r2 // WIN == c2).astype(dtype)
    return eh, ew


def _topk_window_mask(pooled, keep, NH, NW):
    N = NH * NW
    pi = jax.lax.bitcast_convert_type(pooled, jnp.int32)

    def cnt_ge(t):
        return jnp.sum((pi >= t).astype(jnp.int32), keepdims=True).reshape(1, 1)

    def body(_, lohi):
        lo, hi = lohi
        mid = lo + ((hi - lo) // 2) + ((hi - lo) & 1)
        ok = cnt_ge(mid) >= keep
        return (jnp.where(ok, mid, lo), jnp.where(ok, hi, mid - 1))

    lo0 = jnp.zeros((1, 1), jnp.int32)
    hi0 = jnp.full((1, 1), 0x7F800000, jnp.int32)
    vstar, _ = jax.lax.fori_loop(0, 31, body, (lo0, hi0))

    gt = pi > vstar
    eq = pi == vstar
    need = keep - jnp.sum(gt.astype(jnp.int32), keepdims=True).reshape(1, 1)

    flat = (jax.lax.broadcasted_iota(jnp.int32, (NH, NW), 0) * NW
            + jax.lax.broadcasted_iota(jnp.int32, (NH, NW), 1))

    def cnt_eq_lt(m):
        return jnp.sum((eq & (flat < m)).astype(jnp.int32),
                       keepdims=True).reshape(1, 1)

    def body2(_, lohi):
        lo, hi = lohi
        mid = (lo + hi) // 2
        ok = cnt_eq_lt(mid) >= need
        return (jnp.where(ok, lo, mid + 1), jnp.where(ok, mid, hi))

    lo0 = jnp.full((1, 1), 1, jnp.int32)
    hi0 = jnp.full((1, 1), N, jnp.int32)
    mcut, _ = jax.lax.fori_loop(0, 10, body2, (lo0, hi0))

    return (gt | (eq & (flat < mcut))).astype(jnp.float32)


def _fused_kernel(x_hbm, o_hbm, wm_ref, tm_ref,
                  buf, isem, osem, *, BT, C, H, W, NH, NW, keep, ns):
    bt = pl.program_id(0)

    def in_copy(frame, s):
        return pltpu.make_async_copy(
            x_hbm.at[frame, s * 16:(s + 1) * 16], buf.at[s], isem.at[s])

    def out_copy(frame, s):
        return pltpu.make_async_copy(
            buf.at[s], o_hbm.at[frame, s * 16:(s + 1) * 16], osem.at[s])

    @pl.when(bt == 0)
    def _():
        for s in range(ns):
            in_copy(0, s).start()

    in_copy(bt, 0).wait()
    e = jnp.sum(jnp.abs(buf[0]), axis=0)
    for s in range(1, ns):
        in_copy(bt, s).wait()
        e = e + jnp.sum(jnp.abs(buf[s]), axis=0)
    energy = e / jnp.float32(C)

    eh, ew = _pool_mats(H, W, NH, NW)
    wsum = jax.lax.dot_general(
        energy, ew, (((1,), (0,)), ((), ())),
        precision=jax.lax.Precision.HIGHEST,
        preferred_element_type=jnp.float32)
    hsum = jax.lax.dot_general(
        eh, wsum, (((1,), (0,)), ((), ())),
        precision=jax.lax.Precision.HIGHEST,
        preferred_element_type=jnp.float32)
    pooled = hsum / jnp.float32(WIN * WIN)

    wmask = _topk_window_mask(pooled, keep, NH, NW)
    wm_ref[0] = wmask
    t1 = jax.lax.dot_general(
        eh, wmask, (((0,), (0,)), ((), ())),
        preferred_element_type=jnp.float32)
    tmask = jax.lax.dot_general(
        t1, ew, (((1,), (1,)), ((), ())),
        preferred_element_type=jnp.float32)
    tm_ref[0] = tmask

    for s in range(ns):
        buf[s] = buf[s] * tmask[None, :, :]
        out_copy(bt, s).start()

    for s in range(ns):
        out_copy(bt, s).wait()

        @pl.when(bt + 1 < BT)
        def _(s=s):
            in_copy(bt + 1, s).start()


def kernel(x):
    B, T, C, H, W = x.shape
    assert H % WIN == 0 and W % WIN == 0 and C % 16 == 0
    NH, NW = H // WIN, W // WIN
    N = NH * NW
    keep = min(max(MIN_KEEP, int(N * KEEP_RATIO)), N)
    BT = B * T
    x4 = x.reshape(BT, C, H, W)

    ns = C // 16
    out, wm, tm = pl.pallas_call(
        functools.partial(_fused_kernel, BT=BT, C=C, H=H, W=W,
                          NH=NH, NW=NW, keep=keep, ns=ns),
        grid=(BT,),
        in_specs=[pl.BlockSpec(memory_space=_ANY)],
        out_specs=[
            pl.BlockSpec(memory_space=_ANY),
            pl.BlockSpec((1, NH, NW), lambda bt: (bt, 0, 0)),
            pl.BlockSpec((1, H, W), lambda bt: (bt, 0, 0)),
        ],
        out_shape=[
            jax.ShapeDtypeStruct((BT, C, H, W), jnp.float32),
            jax.ShapeDtypeStruct((BT, NH, NW), jnp.float32),
            jax.ShapeDtypeStruct((BT, H, W), jnp.float32),
        ],
        scratch_shapes=[
            pltpu.VMEM((ns, 16, H, W), jnp.float32),
            pltpu.SemaphoreType.DMA((ns,)),
            pltpu.SemaphoreType.DMA((ns,)),
        ],
        compiler_params=pltpu.CompilerParams(
            dimension_semantics=("arbitrary",),
            vmem_limit_bytes=60 * 1024 * 1024),
    )(x4)

    pruned = out.reshape(B, T, C, H, W)
    token_mask = tm.reshape(B, T, H, W).astype(jnp.bool_)
    window_mask = wm.reshape(B, T, NH, NW).astype(jnp.bool_)
    return (pruned, token_mask, window_mask)
